# Initial kernel scaffold; baseline (speedup 1.0000x reference)
import jax
import jax.numpy as jnp
from jax.experimental import pallas as pl


def kernel(x, W, scale, per_expert_scale):
    raise NotImplementedError("write your pallas kernel here")


# speedup vs baseline: 10.3147x; 10.3147x over previous
import functools

import jax
import jax.numpy as jnp
from jax.experimental import pallas as pl
from jax.experimental.pallas import tpu as pltpu

D_MODEL = 2816
N_EXPERTS = 64
TOP_K = 8
RMS_EPS = 1e-06
_DSCALE = D_MODEL ** -0.5


def _router_kernel(x_ref, w_ref, scale_ref, pes_ref, out_ref):
    x = x_ref[...]
    v = jnp.mean(x * x, axis=-1, keepdims=True)
    h = x * jax.lax.rsqrt(v + RMS_EPS)
    h = h * scale_ref[...] * _DSCALE
    scores = jax.lax.dot_general(
        h, w_ref[...], (((1,), (1,)), ((), ())),
        preferred_element_type=jnp.float32)

    neg = jnp.float32(-jnp.inf)
    iota = jax.lax.broadcasted_iota(jnp.int32, scores.shape, 1)
    remaining = scores
    sel = jnp.zeros(scores.shape, dtype=jnp.bool_)
    for _ in range(TOP_K):
        m = jnp.max(remaining, axis=-1, keepdims=True)
        ismax = remaining == m
        amin = jnp.min(jnp.where(ismax, iota, N_EXPERTS), axis=-1, keepdims=True)
        first = iota == amin
        sel = jnp.logical_or(sel, first)
        remaining = jnp.where(first, neg, remaining)

    rowmax = jnp.max(scores, axis=-1, keepdims=True)
    e = jnp.where(sel, jnp.exp(scores - rowmax), 0.0)
    denom = jnp.sum(e, axis=-1, keepdims=True)
    out_ref[...] = e * (pes_ref[...] / denom)


@functools.partial(jax.jit, static_argnames=("block_t",))
def _run(x2d, W, scale, per_expert_scale, block_t):
    n_tok = x2d.shape[0]
    grid = (n_tok // block_t,)
    return pl.pallas_call(
        _router_kernel,
        grid=grid,
        in_specs=[
            pl.BlockSpec((block_t, D_MODEL), lambda i: (i, 0)),
            pl.BlockSpec((N_EXPERTS, D_MODEL), lambda i: (0, 0)),
            pl.BlockSpec((1, D_MODEL), lambda i: (0, 0)),
            pl.BlockSpec((1, N_EXPERTS), lambda i: (0, 0)),
        ],
        out_specs=pl.BlockSpec((block_t, N_EXPERTS), lambda i: (i, 0)),
        out_shape=jax.ShapeDtypeStruct((n_tok, N_EXPERTS), jnp.float32),
        compiler_params=pltpu.CompilerParams(
            dimension_semantics=("arbitrary",),
        ),
    )(x2d, W, scale.reshape(1, D_MODEL), per_expert_scale.reshape(1, N_EXPERTS))


def kernel(x, W, scale, per_expert_scale):
    b, t, d = x.shape
    x2d = x.reshape(b * t, d)
    out = _run(x2d, W, scale, per_expert_scale, block_t=512)
    return out.reshape(b, t, N_EXPERTS)


# speedup vs baseline: 14.0502x; 1.3622x over previous
import functools

import jax
import jax.numpy as jnp
from jax.experimental import pallas as pl
from jax.experimental.pallas import tpu as pltpu

D_MODEL = 2816
N_EXPERTS = 64
TOP_K = 8
RMS_EPS = 1e-06
_DSCALE = D_MODEL ** -0.5


def _router_kernel(x_ref, w_ref, scale_ref, pes_ref, out_ref):
    x = x_ref[...]
    v = jnp.mean(x * x, axis=-1, keepdims=True)
    h = x * jax.lax.rsqrt(v + RMS_EPS)
    h = h * scale_ref[...] * _DSCALE
    s = jax.lax.dot_general(
        w_ref[...], h, (((1,), (1,)), ((), ())),
        preferred_element_type=jnp.float32)

    iota = jax.lax.broadcasted_iota(jnp.int32, s.shape, 0)
    neg = jnp.float32(-jnp.inf)
    remaining = s
    sel = jnp.zeros(s.shape, dtype=jnp.bool_)
    rowmax = None
    for it in range(TOP_K):
        m = jnp.max(remaining, axis=0, keepdims=True)
        if it == 0:
            rowmax = m
        ismax = remaining == m
        amin = jnp.min(jnp.where(ismax, iota, N_EXPERTS), axis=0, keepdims=True)
        first = iota == amin
        sel = jnp.logical_or(sel, first)
        remaining = jnp.where(first, neg, remaining)

    e = jnp.where(sel, jnp.exp(s - rowmax), 0.0)
    denom = jnp.sum(e, axis=0, keepdims=True)
    out_ref[...] = (e / denom) * pes_ref[...][:, 0:1]


@functools.partial(jax.jit, static_argnames=("block_t",))
def _run(x2d, W, scale, per_expert_scale, block_t):
    n_tok = x2d.shape[0]
    grid = (n_tok // block_t,)
    pes2 = jnp.broadcast_to(
        per_expert_scale.reshape(N_EXPERTS, 1), (N_EXPERTS, 128))
    out_t = pl.pallas_call(
        _router_kernel,
        grid=grid,
        in_specs=[
            pl.BlockSpec((block_t, D_MODEL), lambda i: (i, 0)),
            pl.BlockSpec((N_EXPERTS, D_MODEL), lambda i: (0, 0)),
            pl.BlockSpec((1, D_MODEL), lambda i: (0, 0)),
            pl.BlockSpec((N_EXPERTS, 128), lambda i: (0, 0)),
        ],
        out_specs=pl.BlockSpec((N_EXPERTS, block_t), lambda i: (0, i)),
        out_shape=jax.ShapeDtypeStruct((N_EXPERTS, n_tok), jnp.float32),
        compiler_params=pltpu.CompilerParams(
            dimension_semantics=("arbitrary",),
        ),
    )(x2d, W, scale.reshape(1, D_MODEL), pes2)
    return out_t


def kernel(x, W, scale, per_expert_scale):
    b, t, d = x.shape
    x2d = x.reshape(b * t, d)
    out_t = _run(x2d, W, scale, per_expert_scale, block_t=512)
    return out_t.T.reshape(b, t, N_EXPERTS)
